# Initial kernel scaffold; baseline (speedup 1.0000x reference)
import jax
import jax.numpy as jnp
from jax.experimental import pallas as pl


def kernel(x, edge_index, batch, W1, b1, W3, b3, Wg, bg):
    raise NotImplementedError("write your pallas kernel here")


# speedup vs baseline: 14.5545x; 14.5545x over previous
import dataclasses
import functools

import jax
import jax.numpy as jnp
from jax import lax
from jax.experimental import pallas as pl
from jax.experimental.pallas import tpu as pltpu
from jax.experimental.pallas import tpu_sc as plsc

N = 10000
E = 320000
D = 128

NC = 2
NS = 16
NW = NC * NS
L = 16
CHUNK = 128
CHUNKS_PER_W = -(-E // (NW * CHUNK))
E_PAD = NW * CHUNKS_PER_W * CHUNK
N_PAD = 10240
ROWS_PER_SUB = N_PAD // NS

_mesh = plsc.VectorSubcoreMesh(core_axis_name="c", subcore_axis_name="s")

_cp = pltpu.CompilerParams()
if "needs_layout_passes" in pltpu.CompilerParams.__dataclass_fields__:
    _cp = dataclasses.replace(_cp, needs_layout_passes=False)


@functools.partial(
    pl.kernel,
    out_type=jax.ShapeDtypeStruct((NW * N_PAD,), jnp.float32),
    mesh=_mesh,
    scratch_types=[
        pltpu.VMEM((CHUNK,), jnp.int32),
        pltpu.VMEM((N_PAD,), jnp.float32),
    ],
    compiler_params=_cp,
)
def _hist_kernel(dst_hbm, out_hbm, idx_v, hist_v):
    c = lax.axis_index("c")
    s = lax.axis_index("s")
    wid = s * NC + c

    @pl.loop(0, N_PAD // L)
    def _(i):
        hist_v[pl.ds(i * L, L)] = jnp.zeros((L,), jnp.float32)

    ones = jnp.ones((L,), jnp.float32)

    @pl.loop(0, CHUNKS_PER_W)
    def _(it):
        base = (wid * CHUNKS_PER_W + it) * CHUNK
        pltpu.sync_copy(dst_hbm.at[pl.ds(base, CHUNK)], idx_v)

        @pl.loop(0, CHUNK // L)
        def _(j):
            idx = idx_v[pl.ds(j * L, L)]
            plsc.addupdate_scatter(hist_v, [idx], ones)

    pltpu.sync_copy(hist_v, out_hbm.at[pl.ds(wid * N_PAD, N_PAD)])


@functools.partial(
    pl.kernel,
    out_type=jax.ShapeDtypeStruct((NC, N_PAD, D), jnp.float32),
    mesh=_mesh,
    scratch_types=[
        pltpu.VMEM((CHUNK,), jnp.int32),
        pltpu.VMEM((CHUNK,), jnp.int32),
        pltpu.VMEM((CHUNK, D), jnp.float32),
        pltpu.VMEM_SHARED((N_PAD, D), jnp.float32),
        pltpu.SemaphoreType.DMA,
    ],
)
def _scatter_kernel(u_hbm, src_hbm, dst_hbm, z128_hbm, out_hbm,
                    sidx_v, didx_v, rows_v, acc_sh, sem):
    c = lax.axis_index("c")
    s = lax.axis_index("s")
    wid = s * NC + c
    pltpu.sync_copy(z128_hbm.at[pl.ds(s * ROWS_PER_SUB, ROWS_PER_SUB)],
                    acc_sh.at[pl.ds(s * ROWS_PER_SUB, ROWS_PER_SUB)])
    plsc.subcore_barrier()

    @pl.loop(0, CHUNKS_PER_W)
    def _(it):
        base = (wid * CHUNKS_PER_W + it) * CHUNK
        pltpu.sync_copy(src_hbm.at[pl.ds(base, CHUNK)], sidx_v)
        pltpu.sync_copy(dst_hbm.at[pl.ds(base, CHUNK)], didx_v)
        pltpu.async_copy(u_hbm.at[sidx_v], rows_v, sem).wait()
        pltpu.sync_copy(rows_v, acc_sh.at[didx_v], add=True)

    plsc.subcore_barrier()
    pltpu.sync_copy(acc_sh.at[pl.ds(s * ROWS_PER_SUB, ROWS_PER_SUB)],
                    out_hbm.at[c, pl.ds(s * ROWS_PER_SUB, ROWS_PER_SUB)])


_MM_BLK = 1024


def _mm_body(x_ref, w1_ref, w3_ref, wg_ref, b1_ref, b3_ref, deg_ref,
             u_ref, dis_ref, wc_ref, rb_ref):
    @pl.when(pl.program_id(0) == 0)
    def _():
        w13 = lax.dot_general(w1_ref[...], w3_ref[...], (((1,), (0,)), ((), ())),
                              precision=lax.Precision.HIGHEST)
        wc_ref[...] = lax.dot_general(w13, wg_ref[...], (((1,), (0,)), ((), ())),
                                      precision=lax.Precision.HIGHEST)
        rb13 = lax.dot_general(b1_ref[...], w3_ref[...], (((1,), (0,)), ((), ())),
                               precision=lax.Precision.HIGHEST) + b3_ref[...]
        rb_ref[...] = lax.dot_general(rb13, wg_ref[...], (((1,), (0,)), ((), ())),
                                      precision=lax.Precision.HIGHEST)

    hw = lax.dot_general(x_ref[...], wc_ref[...], (((1,), (0,)), ((), ())),
                         precision=lax.Precision.HIGHEST) + rb_ref[...]
    deg = jnp.sum(deg_ref[...], axis=0) + 1.0
    dis = lax.rsqrt(deg)
    dis_ref[...] = dis
    u_ref[...] = hw * dis[:, None]


def _run_mm(x_p, W1, W3, Wg, b1r, b3r, deg32):
    grid = (N_PAD // _MM_BLK,)
    return pl.pallas_call(
        _mm_body,
        grid=grid,
        in_specs=[
            pl.BlockSpec((_MM_BLK, D), lambda i: (i, 0)),
            pl.BlockSpec((D, D), lambda i: (0, 0)),
            pl.BlockSpec((D, D), lambda i: (0, 0)),
            pl.BlockSpec((D, D), lambda i: (0, 0)),
            pl.BlockSpec((1, D), lambda i: (0, 0)),
            pl.BlockSpec((1, D), lambda i: (0, 0)),
            pl.BlockSpec((NW, _MM_BLK), lambda i: (0, i)),
        ],
        out_specs=[
            pl.BlockSpec((_MM_BLK, D), lambda i: (i, 0)),
            pl.BlockSpec((_MM_BLK,), lambda i: (i,)),
        ],
        out_shape=[
            jax.ShapeDtypeStruct((N_PAD, D), jnp.float32),
            jax.ShapeDtypeStruct((N_PAD,), jnp.float32),
        ],
        scratch_shapes=[pltpu.VMEM((D, D), jnp.float32),
                        pltpu.VMEM((1, D), jnp.float32)],
    )(x_p, W1, W3, Wg, b1r, b3r, deg32)


def _fin_body(p_ref, u_ref, dis_ref, bg_ref, o_ref):
    acc = p_ref[0] + p_ref[1] + u_ref[...]
    o_ref[...] = acc * dis_ref[...][:, None] + bg_ref[...]


def _run_final(partials, u, dis, bgr):
    grid = (N_PAD // _MM_BLK,)
    return pl.pallas_call(
        _fin_body,
        grid=grid,
        in_specs=[
            pl.BlockSpec((NC, _MM_BLK, D), lambda i: (0, i, 0)),
            pl.BlockSpec((_MM_BLK, D), lambda i: (i, 0)),
            pl.BlockSpec((_MM_BLK,), lambda i: (i,)),
            pl.BlockSpec((1, D), lambda i: (0, 0)),
        ],
        out_specs=pl.BlockSpec((_MM_BLK, D), lambda i: (i, 0)),
        out_shape=jax.ShapeDtypeStruct((N_PAD, D), jnp.float32),
    )(partials, u, dis, bgr)


def kernel(x, edge_index, batch, W1, b1, W3, b3, Wg, bg):
    del batch
    src = edge_index[0]
    dst = edge_index[1]
    pad = E_PAD - E
    src_p = jnp.concatenate([src, jnp.full((pad,), N, jnp.int32)])
    dst_p = jnp.concatenate([dst, jnp.full((pad,), N, jnp.int32)])
    x_p = jnp.zeros((N_PAD, D), jnp.float32).at[:N].set(x)
    z128 = jnp.zeros((N_PAD, D), jnp.float32)
    b1r = b1.reshape(1, D)
    b3r = b3.reshape(1, D)
    bgr = bg.reshape(1, D)

    deg_flat = _hist_kernel(dst_p)
    deg32 = deg_flat.reshape(NW, N_PAD)
    u, dis = _run_mm(x_p, W1, W3, Wg, b1r, b3r, deg32)
    partials = _scatter_kernel(u, src_p, dst_p, z128)
    return _run_final(partials, u, dis, bgr)[:N]


# speedup vs baseline: 16.1782x; 1.1116x over previous
import dataclasses
import functools

import jax
import jax.numpy as jnp
from jax import lax
from jax.experimental import pallas as pl
from jax.experimental.pallas import tpu as pltpu
from jax.experimental.pallas import tpu_sc as plsc

N = 10000
E = 320000
D = 128

NC = 2
NS = 16
NW = NC * NS
L = 16
CHUNK = 128
CHUNKS_PER_W = 80
E_PER_W = CHUNKS_PER_W * CHUNK
E_PAD = NW * E_PER_W
N_PAD = 10240
ROWS_PER_SUB = N_PAD // NS

_mesh = plsc.VectorSubcoreMesh(core_axis_name="c", subcore_axis_name="s")

_cp = pltpu.CompilerParams()
if "needs_layout_passes" in pltpu.CompilerParams.__dataclass_fields__:
    _cp = dataclasses.replace(_cp, needs_layout_passes=False)


@functools.partial(
    pl.kernel,
    out_type=jax.ShapeDtypeStruct((NW * N_PAD,), jnp.float32),
    mesh=_mesh,
    scratch_types=[
        pltpu.VMEM((E_PER_W,), jnp.int32),
        pltpu.VMEM((N_PAD,), jnp.float32),
    ],
    compiler_params=_cp,
)
def _hist_kernel(dst_hbm, out_hbm, idx_v, hist_v):
    c = lax.axis_index("c")
    s = lax.axis_index("s")
    wid = s * NC + c
    pltpu.sync_copy(dst_hbm.at[pl.ds(wid * E_PER_W, E_PER_W)], idx_v)

    @pl.loop(0, N_PAD // L)
    def _(i):
        hist_v[pl.ds(i * L, L)] = jnp.zeros((L,), jnp.float32)

    ones = jnp.ones((L,), jnp.float32)

    @pl.loop(0, E_PER_W // L)
    def _(t):
        idx = idx_v[pl.ds(t * L, L)]
        plsc.addupdate_scatter(hist_v, [idx], ones)

    pltpu.sync_copy(hist_v, out_hbm.at[pl.ds(wid * N_PAD, N_PAD)])


@functools.partial(
    pl.kernel,
    out_type=jax.ShapeDtypeStruct((NC, N_PAD, D), jnp.float32),
    mesh=_mesh,
    scratch_types=[
        pltpu.VMEM((2, CHUNK), jnp.int32),
        pltpu.VMEM((2, CHUNK), jnp.int32),
        pltpu.VMEM((2, CHUNK), jnp.int32),
        pltpu.VMEM((2, CHUNK), jnp.int32),
        pltpu.VMEM((CHUNK, D), jnp.float32),
        pltpu.VMEM((CHUNK, D), jnp.float32),
        pltpu.VMEM_SHARED((N_PAD, D), jnp.float32),
        pltpu.SemaphoreType.DMA,
        pltpu.SemaphoreType.DMA,
        pltpu.SemaphoreType.DMA,
        pltpu.SemaphoreType.DMA,
        pltpu.SemaphoreType.DMA,
        pltpu.SemaphoreType.DMA,
    ],
)
def _scatter_kernel(u_hbm, edges_hbm, z128_hbm, out_hbm,
                    e0, e1, e2, e3, rows0, rows1, acc_sh,
                    semE0, semE1, semE2, semE3, semG0, semG1):
    c = lax.axis_index("c")
    s = lax.axis_index("s")
    wid = s * NC + c
    g0 = wid * CHUNKS_PER_W
    ebufs = (e0, e1, e2, e3)
    esems = (semE0, semE1, semE2, semE3)

    def idx_issue(j, k):
        pltpu.async_copy(edges_hbm.at[pl.ds(2 * (g0 + j), 2)], ebufs[k], esems[k])

    def idx_wait(j, k):
        pltpu.make_async_copy(edges_hbm.at[pl.ds(2 * (g0 + j), 2)],
                              ebufs[k], esems[k]).wait()

    def gather_issue(k, rbuf, sem):
        pltpu.async_copy(u_hbm.at[ebufs[k].at[0]], rbuf, sem)

    def gather_wait(k, rbuf, sem):
        pltpu.make_async_copy(u_hbm.at[ebufs[k].at[0]], rbuf, sem).wait()

    def scatter(k, rbuf):
        pltpu.sync_copy(rbuf, acc_sh.at[ebufs[k].at[1]], add=True)

    idx_issue(0, 0)
    idx_issue(1, 1)
    idx_issue(2, 2)
    idx_issue(3, 3)
    pltpu.sync_copy(z128_hbm.at[pl.ds(s * ROWS_PER_SUB, ROWS_PER_SUB)],
                    acc_sh.at[pl.ds(s * ROWS_PER_SUB, ROWS_PER_SUB)])
    plsc.subcore_barrier()
    idx_wait(0, 0)
    gather_issue(0, rows0, semG0)

    @pl.loop(0, CHUNKS_PER_W // 4)
    def _(i):
        j = i * 4
        idx_wait(j + 1, 1)
        gather_issue(1, rows1, semG1)
        gather_wait(0, rows0, semG0)
        scatter(0, rows0)

        @pl.when(j + 4 < CHUNKS_PER_W)
        def _():
            idx_issue(j + 4, 0)

        idx_wait(j + 2, 2)
        gather_issue(2, rows0, semG0)
        gather_wait(1, rows1, semG1)
        scatter(1, rows1)

        @pl.when(j + 5 < CHUNKS_PER_W)
        def _():
            idx_issue(j + 5, 1)

        idx_wait(j + 3, 3)
        gather_issue(3, rows1, semG1)
        gather_wait(2, rows0, semG0)
        scatter(2, rows0)

        @pl.when(j + 6 < CHUNKS_PER_W)
        def _():
            idx_issue(j + 6, 2)

        @pl.when(j + 4 < CHUNKS_PER_W)
        def _():
            idx_wait(j + 4, 0)
            gather_issue(0, rows0, semG0)

        gather_wait(3, rows1, semG1)
        scatter(3, rows1)

        @pl.when(j + 7 < CHUNKS_PER_W)
        def _():
            idx_issue(j + 7, 3)

    plsc.subcore_barrier()
    pltpu.sync_copy(acc_sh.at[pl.ds(s * ROWS_PER_SUB, ROWS_PER_SUB)],
                    out_hbm.at[c, pl.ds(s * ROWS_PER_SUB, ROWS_PER_SUB)])


_MM_BLK = 1024


def _mm_body(x_ref, w1_ref, w3_ref, wg_ref, b1_ref, b3_ref,
             hw_ref, wc_ref, rb_ref):
    @pl.when(pl.program_id(0) == 0)
    def _():
        w13 = lax.dot_general(w1_ref[...], w3_ref[...], (((1,), (0,)), ((), ())),
                              precision=lax.Precision.HIGHEST)
        wc_ref[...] = lax.dot_general(w13, wg_ref[...], (((1,), (0,)), ((), ())),
                                      precision=lax.Precision.HIGHEST)
        rb13 = lax.dot_general(b1_ref[...], w3_ref[...], (((1,), (0,)), ((), ())),
                               precision=lax.Precision.HIGHEST) + b3_ref[...]
        rb_ref[...] = lax.dot_general(rb13, wg_ref[...], (((1,), (0,)), ((), ())),
                                      precision=lax.Precision.HIGHEST)

    hw_ref[...] = lax.dot_general(x_ref[...], wc_ref[...], (((1,), (0,)), ((), ())),
                                  precision=lax.Precision.HIGHEST) + rb_ref[...]


def _run_mm(x_p, W1, W3, Wg, b1r, b3r):
    grid = (N_PAD // _MM_BLK,)
    return pl.pallas_call(
        _mm_body,
        grid=grid,
        in_specs=[
            pl.BlockSpec((_MM_BLK, D), lambda i: (i, 0)),
            pl.BlockSpec((D, D), lambda i: (0, 0)),
            pl.BlockSpec((D, D), lambda i: (0, 0)),
            pl.BlockSpec((D, D), lambda i: (0, 0)),
            pl.BlockSpec((1, D), lambda i: (0, 0)),
            pl.BlockSpec((1, D), lambda i: (0, 0)),
        ],
        out_specs=pl.BlockSpec((_MM_BLK, D), lambda i: (i, 0)),
        out_shape=jax.ShapeDtypeStruct((N_PAD, D), jnp.float32),
        scratch_shapes=[pltpu.VMEM((D, D), jnp.float32),
                        pltpu.VMEM((1, D), jnp.float32)],
    )(x_p, W1, W3, Wg, b1r, b3r)


def _scale_body(hw_ref, deg_ref, u_ref, dis_ref):
    deg = jnp.sum(deg_ref[...], axis=0) + 1.0
    dis = lax.rsqrt(deg)
    dis_ref[...] = dis
    u_ref[...] = hw_ref[...] * dis[:, None]


def _run_scale(hw, deg32):
    grid = (N_PAD // _MM_BLK,)
    return pl.pallas_call(
        _scale_body,
        grid=grid,
        in_specs=[
            pl.BlockSpec((_MM_BLK, D), lambda i: (i, 0)),
            pl.BlockSpec((NW, _MM_BLK), lambda i: (0, i)),
        ],
        out_specs=[
            pl.BlockSpec((_MM_BLK, D), lambda i: (i, 0)),
            pl.BlockSpec((_MM_BLK,), lambda i: (i,)),
        ],
        out_shape=[
            jax.ShapeDtypeStruct((N_PAD, D), jnp.float32),
            jax.ShapeDtypeStruct((N_PAD,), jnp.float32),
        ],
    )(hw, deg32)


def _fin_body(p_ref, u_ref, dis_ref, bg_ref, o_ref):
    acc = p_ref[0] + p_ref[1] + u_ref[...]
    o_ref[...] = acc * dis_ref[...][:, None] + bg_ref[...]


def _run_final(partials, u, dis, bgr):
    grid = (N_PAD // _MM_BLK,)
    return pl.pallas_call(
        _fin_body,
        grid=grid,
        in_specs=[
            pl.BlockSpec((NC, _MM_BLK, D), lambda i: (0, i, 0)),
            pl.BlockSpec((_MM_BLK, D), lambda i: (i, 0)),
            pl.BlockSpec((_MM_BLK,), lambda i: (i,)),
            pl.BlockSpec((1, D), lambda i: (0, 0)),
        ],
        out_specs=pl.BlockSpec((_MM_BLK, D), lambda i: (i, 0)),
        out_shape=jax.ShapeDtypeStruct((N_PAD, D), jnp.float32),
    )(partials, u, dis, bgr)


def kernel(x, edge_index, batch, W1, b1, W3, b3, Wg, bg):
    del batch
    src = edge_index[0]
    dst = edge_index[1]
    pad = E_PAD - E
    src_p = jnp.concatenate([src, jnp.full((pad,), N, jnp.int32)])
    dst_p = jnp.concatenate([dst, jnp.full((pad,), N, jnp.int32)])
    src2 = src_p.reshape(E_PAD // CHUNK, CHUNK)
    dst2 = dst_p.reshape(E_PAD // CHUNK, CHUNK)
    edges2 = jnp.stack([src2, dst2], axis=1).reshape(-1, CHUNK)
    x_p = jnp.zeros((N_PAD, D), jnp.float32).at[:N].set(x)
    z128 = jnp.zeros((N_PAD, D), jnp.float32)
    b1r = b1.reshape(1, D)
    b3r = b3.reshape(1, D)
    bgr = bg.reshape(1, D)

    deg_flat = _hist_kernel(dst_p)
    deg32 = deg_flat.reshape(NW, N_PAD)
    hw = _run_mm(x_p, W1, W3, Wg, b1r, b3r)
    u, dis = _run_scale(hw, deg32)
    partials = _scatter_kernel(u, edges2, z128)
    return _run_final(partials, u, dis, bgr)[:N]


# speedup vs baseline: 16.2109x; 1.0020x over previous
import dataclasses
import functools

import jax
import jax.numpy as jnp
from jax import lax
from jax.experimental import pallas as pl
from jax.experimental.pallas import tpu as pltpu
from jax.experimental.pallas import tpu_sc as plsc

N = 10000
E = 320000
D = 128

NC = 2
NS = 16
NW = NC * NS
L = 16
CHUNK = 128
CHUNKS_PER_W = 80
E_PER_W = CHUNKS_PER_W * CHUNK
E_PAD = NW * E_PER_W
N_PAD = 10240
ROWS_PER_SUB = N_PAD // NS

_mesh = plsc.VectorSubcoreMesh(core_axis_name="c", subcore_axis_name="s")

_cp = pltpu.CompilerParams()
if "needs_layout_passes" in pltpu.CompilerParams.__dataclass_fields__:
    _cp = dataclasses.replace(_cp, needs_layout_passes=False)


@functools.partial(
    pl.kernel,
    out_type=jax.ShapeDtypeStruct((NW * N_PAD,), jnp.float32),
    mesh=_mesh,
    scratch_types=[
        pltpu.VMEM((E_PER_W,), jnp.int32),
        pltpu.VMEM((N_PAD,), jnp.float32),
    ],
    compiler_params=_cp,
)
def _hist_kernel(dst_hbm, out_hbm, idx_v, hist_v):
    c = lax.axis_index("c")
    s = lax.axis_index("s")
    wid = s * NC + c
    pltpu.sync_copy(dst_hbm.at[pl.ds(wid * E_PER_W, E_PER_W)], idx_v)

    @pl.loop(0, N_PAD // L)
    def _(i):
        hist_v[pl.ds(i * L, L)] = jnp.zeros((L,), jnp.float32)

    ones = jnp.ones((L,), jnp.float32)

    @pl.loop(0, E_PER_W // L)
    def _(t):
        idx = idx_v[pl.ds(t * L, L)]
        plsc.addupdate_scatter(hist_v, [idx], ones)

    pltpu.sync_copy(hist_v, out_hbm.at[pl.ds(wid * N_PAD, N_PAD)])


@functools.partial(
    pl.kernel,
    out_type=jax.ShapeDtypeStruct((NC, N_PAD, D), jnp.float32),
    mesh=_mesh,
    scratch_types=[
        pltpu.VMEM((2, CHUNK), jnp.int32),
        pltpu.VMEM((2, CHUNK), jnp.int32),
        pltpu.VMEM((2, CHUNK), jnp.int32),
        pltpu.VMEM((2, CHUNK), jnp.int32),
        pltpu.VMEM((CHUNK, D), jnp.float32),
        pltpu.VMEM((CHUNK, D), jnp.float32),
        pltpu.VMEM_SHARED((N_PAD, D), jnp.float32),
        pltpu.SemaphoreType.DMA,
        pltpu.SemaphoreType.DMA,
        pltpu.SemaphoreType.DMA,
        pltpu.SemaphoreType.DMA,
        pltpu.SemaphoreType.DMA,
        pltpu.SemaphoreType.DMA,
    ],
)
def _scatter_kernel(u_hbm, edges_hbm, z128_hbm, out_hbm,
                    e0, e1, e2, e3, rows0, rows1, acc_sh,
                    semE0, semE1, semE2, semE3, semG0, semG1):
    c = lax.axis_index("c")
    s = lax.axis_index("s")
    wid = s * NC + c
    g0 = wid * CHUNKS_PER_W
    ebufs = (e0, e1, e2, e3)
    esems = (semE0, semE1, semE2, semE3)

    def idx_issue(j, k):
        pltpu.async_copy(edges_hbm.at[pl.ds(2 * (g0 + j), 2)], ebufs[k], esems[k])

    def idx_wait(j, k):
        pltpu.make_async_copy(edges_hbm.at[pl.ds(2 * (g0 + j), 2)],
                              ebufs[k], esems[k]).wait()

    def gather_issue(k, rbuf, sem):
        pltpu.async_copy(u_hbm.at[ebufs[k].at[0]], rbuf, sem)

    def gather_wait(k, rbuf, sem):
        pltpu.make_async_copy(u_hbm.at[ebufs[k].at[0]], rbuf, sem).wait()

    def scatter(k, rbuf):
        pltpu.sync_copy(rbuf, acc_sh.at[ebufs[k].at[1]], add=True)

    idx_issue(0, 0)
    idx_issue(1, 1)
    idx_issue(2, 2)
    idx_issue(3, 3)
    pltpu.sync_copy(z128_hbm.at[pl.ds(s * ROWS_PER_SUB, ROWS_PER_SUB)],
                    acc_sh.at[pl.ds(s * ROWS_PER_SUB, ROWS_PER_SUB)])
    plsc.subcore_barrier()
    idx_wait(0, 0)
    gather_issue(0, rows0, semG0)

    @pl.loop(0, CHUNKS_PER_W // 4)
    def _(i):
        j = i * 4
        idx_wait(j + 1, 1)
        gather_issue(1, rows1, semG1)
        gather_wait(0, rows0, semG0)
        scatter(0, rows0)

        @pl.when(j + 4 < CHUNKS_PER_W)
        def _():
            idx_issue(j + 4, 0)

        idx_wait(j + 2, 2)
        gather_issue(2, rows0, semG0)
        gather_wait(1, rows1, semG1)
        scatter(1, rows1)

        @pl.when(j + 5 < CHUNKS_PER_W)
        def _():
            idx_issue(j + 5, 1)

        idx_wait(j + 3, 3)
        gather_issue(3, rows1, semG1)
        gather_wait(2, rows0, semG0)
        scatter(2, rows0)

        @pl.when(j + 6 < CHUNKS_PER_W)
        def _():
            idx_issue(j + 6, 2)

        @pl.when(j + 4 < CHUNKS_PER_W)
        def _():
            idx_wait(j + 4, 0)
            gather_issue(0, rows0, semG0)

        gather_wait(3, rows1, semG1)
        scatter(3, rows1)

        @pl.when(j + 7 < CHUNKS_PER_W)
        def _():
            idx_issue(j + 7, 3)

    plsc.subcore_barrier()
    pltpu.sync_copy(acc_sh.at[pl.ds(s * ROWS_PER_SUB, ROWS_PER_SUB)],
                    out_hbm.at[c, pl.ds(s * ROWS_PER_SUB, ROWS_PER_SUB)])


_MM_BLK = 1024


def _dot3(a, b):
    dn = (((1,), (0,)), ((), ()))
    f32 = jnp.float32
    ah = a.astype(jnp.bfloat16)
    al = (a - ah.astype(f32)).astype(jnp.bfloat16)
    bh = b.astype(jnp.bfloat16)
    bl = (b - bh.astype(f32)).astype(jnp.bfloat16)
    hh = lax.dot_general(ah, bh, dn, preferred_element_type=f32)
    hl = lax.dot_general(ah, bl, dn, preferred_element_type=f32)
    lh = lax.dot_general(al, bh, dn, preferred_element_type=f32)
    return hh + hl + lh


def _mm_body(x_ref, w1_ref, w3_ref, wg_ref, b1_ref, b3_ref,
             hw_ref, wc_ref, rb_ref):
    @pl.when(pl.program_id(0) == 0)
    def _():
        w13 = _dot3(w1_ref[...], w3_ref[...])
        wc_ref[...] = _dot3(w13, wg_ref[...])
        rb13 = _dot3(b1_ref[...], w3_ref[...]) + b3_ref[...]
        rb_ref[...] = _dot3(rb13, wg_ref[...])

    hw_ref[...] = _dot3(x_ref[...], wc_ref[...]) + rb_ref[...]


def _run_mm(x_p, W1, W3, Wg, b1r, b3r):
    grid = (N_PAD // _MM_BLK,)
    return pl.pallas_call(
        _mm_body,
        grid=grid,
        in_specs=[
            pl.BlockSpec((_MM_BLK, D), lambda i: (i, 0)),
            pl.BlockSpec((D, D), lambda i: (0, 0)),
            pl.BlockSpec((D, D), lambda i: (0, 0)),
            pl.BlockSpec((D, D), lambda i: (0, 0)),
            pl.BlockSpec((1, D), lambda i: (0, 0)),
            pl.BlockSpec((1, D), lambda i: (0, 0)),
        ],
        out_specs=pl.BlockSpec((_MM_BLK, D), lambda i: (i, 0)),
        out_shape=jax.ShapeDtypeStruct((N_PAD, D), jnp.float32),
        scratch_shapes=[pltpu.VMEM((D, D), jnp.float32),
                        pltpu.VMEM((1, D), jnp.float32)],
    )(x_p, W1, W3, Wg, b1r, b3r)


def _scale_body(hw_ref, deg_ref, u_ref, dis_ref):
    deg = jnp.sum(deg_ref[...], axis=0) + 1.0
    dis = lax.rsqrt(deg)
    dis_ref[...] = dis
    u_ref[...] = hw_ref[...] * dis[:, None]


def _run_scale(hw, deg32):
    grid = (N_PAD // _MM_BLK,)
    return pl.pallas_call(
        _scale_body,
        grid=grid,
        in_specs=[
            pl.BlockSpec((_MM_BLK, D), lambda i: (i, 0)),
            pl.BlockSpec((NW, _MM_BLK), lambda i: (0, i)),
        ],
        out_specs=[
            pl.BlockSpec((_MM_BLK, D), lambda i: (i, 0)),
            pl.BlockSpec((_MM_BLK,), lambda i: (i,)),
        ],
        out_shape=[
            jax.ShapeDtypeStruct((N_PAD, D), jnp.float32),
            jax.ShapeDtypeStruct((N_PAD,), jnp.float32),
        ],
    )(hw, deg32)


def _fin_body(p_ref, u_ref, dis_ref, bg_ref, o_ref):
    acc = p_ref[0] + p_ref[1] + u_ref[...]
    o_ref[...] = acc * dis_ref[...][:, None] + bg_ref[...]


def _run_final(partials, u, dis, bgr):
    grid = (N_PAD // _MM_BLK,)
    return pl.pallas_call(
        _fin_body,
        grid=grid,
        in_specs=[
            pl.BlockSpec((NC, _MM_BLK, D), lambda i: (0, i, 0)),
            pl.BlockSpec((_MM_BLK, D), lambda i: (i, 0)),
            pl.BlockSpec((_MM_BLK,), lambda i: (i,)),
            pl.BlockSpec((1, D), lambda i: (0, 0)),
        ],
        out_specs=pl.BlockSpec((_MM_BLK, D), lambda i: (i, 0)),
        out_shape=jax.ShapeDtypeStruct((N_PAD, D), jnp.float32),
    )(partials, u, dis, bgr)


def kernel(x, edge_index, batch, W1, b1, W3, b3, Wg, bg):
    del batch
    src = edge_index[0]
    dst = edge_index[1]
    pad = E_PAD - E
    src_p = jnp.concatenate([src, jnp.full((pad,), N, jnp.int32)])
    dst_p = jnp.concatenate([dst, jnp.full((pad,), N, jnp.int32)])
    src2 = src_p.reshape(E_PAD // CHUNK, CHUNK)
    dst2 = dst_p.reshape(E_PAD // CHUNK, CHUNK)
    edges2 = jnp.stack([src2, dst2], axis=1).reshape(-1, CHUNK)
    x_p = jnp.zeros((N_PAD, D), jnp.float32).at[:N].set(x)
    z128 = jnp.zeros((N_PAD, D), jnp.float32)
    b1r = b1.reshape(1, D)
    b3r = b3.reshape(1, D)
    bgr = bg.reshape(1, D)

    deg_flat = _hist_kernel(dst_p)
    deg32 = deg_flat.reshape(NW, N_PAD)
    hw = _run_mm(x_p, W1, W3, Wg, b1r, b3r)
    u, dis = _run_scale(hw, deg32)
    partials = _scatter_kernel(u, edges2, z128)
    return _run_final(partials, u, dis, bgr)[:N]
